# baseline (speedup 1.0000x reference)
import functools

import jax
import jax.numpy as jnp
from jax import lax
from jax.experimental import pallas as pl
from jax.experimental.pallas import tpu as pltpu
from jax.experimental.pallas import tpu_sc as plsc

_B, _C, _N, _K = 32, 384, 256, 16
_C_OUT = 768
_L = 16
_CW = _C // 2
_JW = _CW // _L
_NCHUNK = 128


def _sc_raw_body(x_hbm, idx_hbm, out_hbm, table_v, out_v, idx_v):
    cid = lax.axis_index("c")
    sid = lax.axis_index("s")
    wid = sid * 2 + cid

    pltpu.sync_copy(x_hbm.at[wid], table_v)
    pltpu.sync_copy(idx_hbm.at[0, wid], idx_v)

    lanes = lax.iota(jnp.int32, _L)

    def chunk_body(ch, _):
        node0 = ch * _NCHUNK

        def node_body(i, _):
            ioff = (node0 + i) * _K
            acc = [None] * _JW
            for r in range(_K):
                rid = plsc.load_gather(
                    idx_v, [jnp.full((_L,), 0, jnp.int32) + (ioff + r)])
                base = rid * _CW + lanes
                for j in range(_JW):
                    w = plsc.load_gather(table_v, [base + j * _L])
                    v = plsc.bitcast(w, jnp.bfloat16)
                    acc[j] = v if r == 0 else jnp.maximum(acc[j], v)
            for j in range(_JW):
                out_v[i, pl.ds(j * 2 * _L, 2 * _L)] = acc[j]
            return ()

        lax.fori_loop(0, _NCHUNK, node_body, ())
        pltpu.sync_copy(out_v, out_hbm.at[pl.ds(wid * _N + node0, _NCHUNK)])
        return ()

    lax.fori_loop(0, _N // _NCHUNK, chunk_body, ())


@functools.cache
def _sc_gather_max():
    mesh = plsc.VectorSubcoreMesh(
        core_axis_name="c", subcore_axis_name="s",
        num_cores=2, num_subcores=16)
    return functools.partial(
        pl.kernel,
        out_type=jax.ShapeDtypeStruct((_B * _N, _C), jnp.bfloat16),
        mesh=mesh,
        scratch_types=[
            pltpu.VMEM((_N * _CW,), jnp.int32),
            pltpu.VMEM((_NCHUNK, _C), jnp.bfloat16),
            pltpu.VMEM((_N * _K,), jnp.int32),
        ],
        compiler_params=pltpu.CompilerParams(
            use_tc_tiling_on_sc=False, needs_layout_passes=False,
            disable_bounds_checks=True),
    )(_sc_raw_body)


def _tc_body(x_ref, xjm_ref, w1_ref, b1_ref, w2a_ref, w2b_ref, b2_ref, out_ref):
    x = x_ref[0]
    h = jnp.maximum(
        lax.dot_general(w1_ref[...], x, (((0,), (0,)), ((), ())),
                        preferred_element_type=jnp.float32) + b1_ref[...],
        0.0)
    xjm = xjm_ref[0]
    pre = (
        lax.dot_general(w2a_ref[...], h, (((0,), (0,)), ((), ())),
                        preferred_element_type=jnp.float32)
        + lax.dot_general(w2b_ref[...], xjm, (((0,), (1,)), ((), ())),
                          preferred_element_type=jnp.float32)
        + b2_ref[...])
    out_ref[0] = jax.nn.sigmoid(jnp.maximum(pre, 0.0))


_tc_dense = pl.pallas_call(
    _tc_body,
    grid=(_B,),
    in_specs=[
        pl.BlockSpec((1, _C, _N), lambda b: (b, 0, 0)),
        pl.BlockSpec((1, _N, _C), lambda b: (b, 0, 0)),
        pl.BlockSpec((_C, _C), lambda b: (0, 0)),
        pl.BlockSpec((_C, 1), lambda b: (0, 0)),
        pl.BlockSpec((_C, _C_OUT), lambda b: (0, 0)),
        pl.BlockSpec((_C, _C_OUT), lambda b: (0, 0)),
        pl.BlockSpec((_C_OUT, 1), lambda b: (0, 0)),
    ],
    out_specs=pl.BlockSpec((1, _C_OUT, _N), lambda b: (b, 0, 0)),
    out_shape=jax.ShapeDtypeStruct((_B, _C_OUT, _N), jnp.float32),
)


def kernel(x, edge_index, W1, b1, W2, b2):
    x_sq = x[:, :, :, 0]
    x_t = jnp.transpose(x_sq, (0, 2, 1)).astype(jnp.bfloat16)
    x_packed = lax.bitcast_convert_type(
        x_t.reshape(_B, _N * _CW, 2), jnp.int32)
    idx = edge_index.astype(jnp.int32).reshape(2, _B, _N * _K)
    xjm = _sc_gather_max()(x_packed, idx)
    out = _tc_dense(x_sq, xjm.reshape(_B, _N, _C), W1,
                    b1.reshape(_C, 1),
                    W2[:_C], W2[_C:].astype(jnp.bfloat16),
                    b2.reshape(_C_OUT, 1))
    return out[:, :, :, None]


# speedup vs baseline: 2.2964x; 2.2964x over previous
import functools

import jax
import jax.numpy as jnp
from jax import lax
from jax.experimental import pallas as pl
from jax.experimental.pallas import tpu as pltpu
from jax.experimental.pallas import tpu_sc as plsc

_B, _C, _N, _K = 32, 384, 256, 16
_C_OUT = 768
_L = 16
_CW = _C // 2
_JW = _CW // _L
_NCHUNK = 128
_SCHUNK = 64


def _sc_raw_body(x_hbm, idx_hbm, out_hbm, table_v, out_v, idx_v, stage_v):
    cid = lax.axis_index("c")
    sid = lax.axis_index("s")
    wid = sid * 2 + cid

    pltpu.sync_copy(idx_hbm.at[0, wid], idx_v)

    def stage_chunk(sc, _):
        row0 = sc * _SCHUNK
        pltpu.sync_copy(x_hbm.at[wid, pl.ds(row0, _SCHUNK)], stage_v)

        def pack_row(i, _):
            for q in range(_C // 32):
                a = stage_v[i, pl.ds(q * 32, _L)]
                b = stage_v[i, pl.ds(q * 32 + _L, _L)]
                w = plsc.bitcast(
                    plsc.pack(a, b, format=plsc.PackFormat.INTERLEAVED),
                    jnp.int32)
                table_v[pl.ds((row0 + i) * _CW + q * _L, _L)] = w
            return ()

        lax.fori_loop(0, _SCHUNK, pack_row, ())
        return ()

    lax.fori_loop(0, _N // _SCHUNK, stage_chunk, ())

    lanes = lax.iota(jnp.int32, _L)

    def chunk_body(ch, _):
        node0 = ch * _NCHUNK

        def node_body(i, _):
            ioff = (node0 + i) * _K
            acc = [None] * _JW
            for r in range(_K):
                rid = plsc.load_gather(
                    idx_v, [jnp.full((_L,), 0, jnp.int32) + (ioff + r)])
                base = rid * _CW + lanes
                for j in range(_JW):
                    w = plsc.load_gather(table_v, [base + j * _L])
                    v = plsc.bitcast(w, jnp.bfloat16)
                    acc[j] = v if r == 0 else jnp.maximum(acc[j], v)
            for j in range(_JW):
                out_v[i, pl.ds(j * 2 * _L, 2 * _L)] = acc[j]
            return ()

        lax.fori_loop(0, _NCHUNK, node_body, ())
        pltpu.sync_copy(out_v, out_hbm.at[pl.ds(wid * _N + node0, _NCHUNK)])
        return ()

    lax.fori_loop(0, _N // _NCHUNK, chunk_body, ())


@functools.cache
def _sc_gather_max():
    mesh = plsc.VectorSubcoreMesh(
        core_axis_name="c", subcore_axis_name="s",
        num_cores=2, num_subcores=16)
    return functools.partial(
        pl.kernel,
        out_type=jax.ShapeDtypeStruct((_B * _N, _C), jnp.bfloat16),
        mesh=mesh,
        scratch_types=[
            pltpu.VMEM((_N * _CW,), jnp.int32),
            pltpu.VMEM((_NCHUNK, _C), jnp.bfloat16),
            pltpu.VMEM((_N * _K,), jnp.int32),
            pltpu.VMEM((_SCHUNK, _C), jnp.float32),
        ],
        compiler_params=pltpu.CompilerParams(
            use_tc_tiling_on_sc=False, needs_layout_passes=False,
            disable_bounds_checks=True),
    )(_sc_raw_body)


def _tc_body(x_ref, xjm_ref, w1_ref, b1_ref, w2a_ref, w2b_ref, b2_ref, out_ref):
    x = x_ref[0]
    h = jnp.maximum(
        lax.dot_general(w1_ref[...], x, (((0,), (0,)), ((), ())),
                        preferred_element_type=jnp.float32) + b1_ref[...],
        0.0)
    xjm = xjm_ref[0]
    pre = (
        lax.dot_general(w2a_ref[...], h, (((0,), (0,)), ((), ())),
                        preferred_element_type=jnp.float32)
        + lax.dot_general(w2b_ref[...], xjm, (((0,), (1,)), ((), ())),
                          preferred_element_type=jnp.float32)
        + b2_ref[...])
    out_ref[0] = jax.nn.sigmoid(jnp.maximum(pre, 0.0))


_tc_dense = pl.pallas_call(
    _tc_body,
    grid=(_B,),
    in_specs=[
        pl.BlockSpec((1, _C, _N), lambda b: (b, 0, 0)),
        pl.BlockSpec((1, _N, _C), lambda b: (b, 0, 0)),
        pl.BlockSpec((_C, _C), lambda b: (0, 0)),
        pl.BlockSpec((_C, 1), lambda b: (0, 0)),
        pl.BlockSpec((_C, _C_OUT), lambda b: (0, 0)),
        pl.BlockSpec((_C, _C_OUT), lambda b: (0, 0)),
        pl.BlockSpec((_C_OUT, 1), lambda b: (0, 0)),
    ],
    out_specs=pl.BlockSpec((1, _C_OUT, _N), lambda b: (b, 0, 0)),
    out_shape=jax.ShapeDtypeStruct((_B, _C_OUT, _N), jnp.float32),
)


_PERM = [
    q * 32 + (t // 2) + (t % 2) * _L
    for q in range(_C // 32) for t in range(32)
]


def kernel(x, edge_index, W1, b1, W2, b2):
    x_sq = x[:, :, :, 0]
    x_t = jnp.transpose(x_sq, (0, 2, 1))
    idx = edge_index.astype(jnp.int32).reshape(2, _B, _N * _K)
    xjm = _sc_gather_max()(x_t, idx)
    w2b = W2[_C:][jnp.array(_PERM)].astype(jnp.bfloat16)
    out = _tc_dense(x_sq, xjm.reshape(_B, _N, _C), W1,
                    b1.reshape(_C, 1), W2[:_C], w2b,
                    b2.reshape(_C_OUT, 1))
    return out[:, :, :, None]


# speedup vs baseline: 2.4067x; 1.0480x over previous
import functools

import jax
import jax.numpy as jnp
from jax import lax
from jax.experimental import pallas as pl
from jax.experimental.pallas import tpu as pltpu
from jax.experimental.pallas import tpu_sc as plsc

_B, _C, _N, _K = 32, 384, 256, 16
_C_OUT = 768
_L = 16
_CW = _C // 2
_JW = _CW // _L
_STRIDE = _CW + 1
_NCHUNK = 128
_CCHUNK = 64


def _sc_raw_body(x_hbm, idx_hbm, out_hbm, table_v, out_v, idx_v, stage_v):
    cid = lax.axis_index("c")
    sid = lax.axis_index("s")
    wid = sid * 2 + cid

    pltpu.sync_copy(idx_hbm.at[0, wid], idx_v)

    lanes = lax.iota(jnp.int32, _L)

    def stage_chunk(sc, _):
        c0 = sc * _CCHUNK
        pltpu.sync_copy(x_hbm.at[wid, pl.ds(c0, _CCHUNK)], stage_v)

        def group_body(g, _):
            rowbase = (g * _L + lanes) * _STRIDE + (c0 // 2)
            for j2 in range(_CCHUNK // 2):
                a = stage_v[2 * j2, pl.ds(g * _L, _L)]
                b = stage_v[2 * j2 + 1, pl.ds(g * _L, _L)]
                w = plsc.bitcast(
                    plsc.pack(a, b, format=plsc.PackFormat.INTERLEAVED),
                    jnp.int32)
                plsc.store_scatter(table_v, [rowbase + j2], w)
            return ()

        lax.fori_loop(0, _N // _L, group_body, ())
        return ()

    lax.fori_loop(0, _C // _CCHUNK, stage_chunk, ())

    def chunk_body(ch, _):
        node0 = ch * _NCHUNK

        def node_body(i, _):
            ioff = (node0 + i) * _K
            acc = [None] * _JW
            for r in range(_K):
                rid = plsc.load_gather(
                    idx_v, [jnp.full((_L,), 0, jnp.int32) + (ioff + r)])
                base = rid * _STRIDE + lanes
                for j in range(_JW):
                    w = plsc.load_gather(table_v, [base + j * _L])
                    v = plsc.bitcast(w, jnp.bfloat16)
                    acc[j] = v if r == 0 else jnp.maximum(acc[j], v)
            for j in range(_JW):
                out_v[i, pl.ds(j * 2 * _L, 2 * _L)] = acc[j]
            return ()

        lax.fori_loop(0, _NCHUNK, node_body, ())
        pltpu.sync_copy(out_v, out_hbm.at[pl.ds(wid * _N + node0, _NCHUNK)])
        return ()

    lax.fori_loop(0, _N // _NCHUNK, chunk_body, ())


@functools.cache
def _sc_gather_max():
    mesh = plsc.VectorSubcoreMesh(
        core_axis_name="c", subcore_axis_name="s",
        num_cores=2, num_subcores=16)
    return functools.partial(
        pl.kernel,
        out_type=jax.ShapeDtypeStruct((_B * _N, _C), jnp.bfloat16),
        mesh=mesh,
        scratch_types=[
            pltpu.VMEM((_N * _STRIDE,), jnp.int32),
            pltpu.VMEM((_NCHUNK, _C), jnp.bfloat16),
            pltpu.VMEM((_N * _K,), jnp.int32),
            pltpu.VMEM((_CCHUNK, _N), jnp.float32),
        ],
        compiler_params=pltpu.CompilerParams(
            use_tc_tiling_on_sc=False, needs_layout_passes=False,
            disable_bounds_checks=True),
    )(_sc_raw_body)


def _tc_body(x_ref, xjm_ref, w1_ref, b1_ref, w2_ref, b2_ref, out_ref):
    x = x_ref[0]
    h = jnp.maximum(
        lax.dot_general(w1_ref[...], x, (((0,), (0,)), ((), ())),
                        preferred_element_type=jnp.float32) + b1_ref[...],
        0.0)
    xjm = xjm_ref[0]
    w2a = w2_ref[pl.ds(0, _C), :]
    w2b = w2_ref[pl.ds(_C, _C), :].astype(jnp.bfloat16)
    pre = (
        lax.dot_general(w2a, h, (((0,), (0,)), ((), ())),
                        preferred_element_type=jnp.float32)
        + lax.dot_general(w2b, xjm, (((0,), (1,)), ((), ())),
                          preferred_element_type=jnp.float32)
        + b2_ref[...])
    out_ref[0] = jax.nn.sigmoid(jnp.maximum(pre, 0.0))


_tc_dense = pl.pallas_call(
    _tc_body,
    grid=(_B,),
    in_specs=[
        pl.BlockSpec((1, _C, _N), lambda b: (b, 0, 0)),
        pl.BlockSpec((1, _N, _C), lambda b: (b, 0, 0)),
        pl.BlockSpec((_C, _C), lambda b: (0, 0)),
        pl.BlockSpec((_C, 1), lambda b: (0, 0)),
        pl.BlockSpec((2 * _C, _C_OUT), lambda b: (0, 0)),
        pl.BlockSpec((_C_OUT, 1), lambda b: (0, 0)),
    ],
    out_specs=pl.BlockSpec((1, _C_OUT, _N), lambda b: (b, 0, 0)),
    out_shape=jax.ShapeDtypeStruct((_B, _C_OUT, _N), jnp.float32),
)


def kernel(x, edge_index, W1, b1, W2, b2):
    x_sq = x.reshape(_B, _C, _N)
    idx = edge_index.astype(jnp.int32).reshape(2, _B, _N * _K)
    xjm = _sc_gather_max()(x_sq, idx)
    out = _tc_dense(x_sq, xjm.reshape(_B, _N, _C), W1,
                    b1.reshape(_C, 1), W2,
                    b2.reshape(_C_OUT, 1))
    return out[:, :, :, None]
